# baseline (speedup 1.0000x reference)
import functools

import jax
import jax.numpy as jnp
from jax import lax
from jax.experimental import pallas as pl
from jax.experimental.pallas import tpu as pltpu
from jax.experimental.pallas import tpu_sc as plsc

VOCAB = 100000
EMBED_DIM = 1024
MAX_SEQ_LEN = 2048
BATCH = 4

NUM_CORES = 2
NUM_SUBCORES = 16
NUM_WORKERS = NUM_CORES * NUM_SUBCORES
POS_PER_WORKER = MAX_SEQ_LEN // NUM_WORKERS
LANES = 16
CHUNK = 8
NUM_CHUNKS = POS_PER_WORKER // CHUNK
NUM_PAIRS = NUM_CHUNKS * 2
PAIR_ROWS = 2 * CHUNK
NBUF = 4


def _embed_body(idx_hbm, msk_hbm, table_hbm, pos_hbm, out_hbm,
                tok_i, msk_i, mmul_x, pbuf, tbuf,
                sem_s, sem_t, sem_p, sem_w):
    wid = lax.axis_index("s") * NUM_CORES + lax.axis_index("c")
    s0 = wid * POS_PER_WORKER

    staged = []
    for p in range(NUM_PAIRS):
        c, h = p // 2, p % 2
        for j in range(2):
            b = 2 * h + j
            src = pl.ds(s0 + c * CHUNK, CHUNK)
            dst = pl.ds(j * CHUNK, CHUNK)
            staged.append(pltpu.async_copy(
                idx_hbm.at[b, src], tok_i.at[p, dst], sem_s))
            staged.append(pltpu.async_copy(
                msk_hbm.at[b, src], msk_i.at[p, dst], sem_s))
    for cp in staged:
        cp.wait()

    def start_gather(p, slot):
        return pltpu.async_copy(
            table_hbm.at[tok_i.at[p]], tbuf.at[slot], sem_t.at[slot])

    def start_writebacks(p, slot):
        c, h = p // 2, p % 2
        cps = []
        for j in range(2):
            cps.append(pltpu.async_copy(
                tbuf.at[slot, pl.ds(j * CHUNK, CHUNK)],
                out_hbm.at[2 * h + j, pl.ds(s0 + c * CHUNK, CHUNK)],
                sem_w.at[slot]))
        return cps

    def start_pos(c, pslot):
        return pltpu.async_copy(
            pos_hbm.at[pl.ds(s0 + c * CHUNK, CHUNK)],
            pbuf.at[pslot], sem_p.at[pslot])

    pos_pending = start_pos(0, 0)
    gathers = [start_gather(0, 0), start_gather(1, 1)] + [None] * (NBUF - 2)
    writebacks = [None] * NBUF

    for p in range(NUM_PAIRS):
        m = msk_i[p, :]
        mmf = jnp.where(m != 0, 0.0, 1.0)
        for lane in range(PAIR_ROWS):
            mmul_x[p, lane, :] = jnp.broadcast_to(mmf[lane], (LANES,))

    for p in range(NUM_PAIRS):
        slot = p % NBUF
        if p % 2 == 0:
            c = p // 2
            pos_pending.wait()
            if c + 1 < NUM_CHUNKS:
                pos_pending = start_pos(c + 1, (c + 1) % 2)
        pslot = (p // 2) % 2
        if p + 2 < NUM_PAIRS:
            pf = (p + 2) % NBUF
            if writebacks[pf] is not None:
                for cp in writebacks[pf]:
                    cp.wait()
                writebacks[pf] = None
            gathers[pf] = start_gather(p + 2, pf)
        gathers[slot].wait()
        gathers[slot] = None

        def row_add(row, carry):
            mra = mmul_x[p, row, :]
            mrb = mmul_x[p, CHUNK + row, :]

            @plsc.parallel_loop(0, EMBED_DIM, step=LANES, unroll=16)
            def vec_add(v):
                vsl = pl.ds(v, LANES)
                pv = pbuf[pslot, row, vsl]
                ta = tbuf[slot, row, vsl]
                tb = tbuf[slot, CHUNK + row, vsl]
                tbuf[slot, row, vsl] = (ta + pv) * mra
                tbuf[slot, CHUNK + row, vsl] = (tb + pv) * mrb

            return carry

        lax.fori_loop(0, CHUNK, row_add, 0)
        writebacks[slot] = start_writebacks(p, slot)

    for wbs in writebacks:
        if wbs is not None:
            for cp in wbs:
                cp.wait()


@functools.partial(jax.jit, donate_argnums=())
def _embed(inputs, masks_i32, token_table, pos_flat):
    mesh = plsc.VectorSubcoreMesh(
        core_axis_name="c", subcore_axis_name="s",
        num_cores=NUM_CORES, num_subcores=NUM_SUBCORES)
    f = pl.kernel(
        _embed_body,
        out_type=jax.ShapeDtypeStruct(
            (BATCH, MAX_SEQ_LEN, EMBED_DIM), jnp.float32),
        mesh=mesh,
        scratch_types=[
            pltpu.VMEM((NUM_PAIRS, PAIR_ROWS), jnp.int32),
            pltpu.VMEM((NUM_PAIRS, PAIR_ROWS), jnp.int32),
            pltpu.VMEM((NUM_PAIRS, PAIR_ROWS, LANES), jnp.float32),
            pltpu.VMEM((2, CHUNK, EMBED_DIM), jnp.float32),
            pltpu.VMEM((NBUF, PAIR_ROWS, EMBED_DIM), jnp.float32),
            pltpu.SemaphoreType.DMA,
            pltpu.SemaphoreType.DMA((NBUF,)),
            pltpu.SemaphoreType.DMA((2,)),
            pltpu.SemaphoreType.DMA((NBUF,)),
        ],
    )
    return f(inputs, masks_i32, token_table, pos_flat)


def kernel(inputs, masks, token_table, pos_embedding):
    idx = inputs.astype(jnp.int32)
    msk = masks.astype(jnp.int32)
    pos_flat = pos_embedding.reshape(MAX_SEQ_LEN, EMBED_DIM)
    return _embed(idx, msk, token_table, pos_flat)


# speedup vs baseline: 1.0430x; 1.0430x over previous
import functools

import jax
import jax.numpy as jnp
from jax import lax
from jax.experimental import pallas as pl
from jax.experimental.pallas import tpu as pltpu
from jax.experimental.pallas import tpu_sc as plsc

VOCAB = 100000
EMBED_DIM = 1024
MAX_SEQ_LEN = 2048
BATCH = 4

NUM_CORES = 2
NUM_SUBCORES = 16
NUM_WORKERS = NUM_CORES * NUM_SUBCORES
POS_PER_WORKER = MAX_SEQ_LEN // NUM_WORKERS
LANES = 16
CHUNK = 8
NUM_CHUNKS = POS_PER_WORKER // CHUNK
NUM_PAIRS = NUM_CHUNKS * 2
PAIR_ROWS = 2 * CHUNK
NBUF = 4


def _embed_body(idx_hbm, msk_hbm, table_hbm, pos_hbm, out_hbm,
                tok_i, msk_i, mmul_x, pbuf, tbuf,
                sem_s, sem_t, sem_p, sem_w):
    wid = lax.axis_index("s") * NUM_CORES + lax.axis_index("c")
    s0 = wid * POS_PER_WORKER

    staged = []
    for p in range(NUM_PAIRS):
        c, h = p // 2, p % 2
        for j in range(2):
            b = 2 * h + j
            src = pl.ds(s0 + c * CHUNK, CHUNK)
            dst = pl.ds(j * CHUNK, CHUNK)
            staged.append(pltpu.async_copy(
                idx_hbm.at[b, src], tok_i.at[p, dst], sem_s))
            staged.append(pltpu.async_copy(
                msk_hbm.at[b, src], msk_i.at[p, dst], sem_s))
    for cp in staged:
        cp.wait()

    def start_gather(p, slot):
        return pltpu.async_copy(
            table_hbm.at[tok_i.at[p]], tbuf.at[slot], sem_t.at[slot])

    def start_writebacks(p, slot):
        c, h = p // 2, p % 2
        cps = []
        for j in range(2):
            cps.append(pltpu.async_copy(
                tbuf.at[slot, pl.ds(j * CHUNK, CHUNK)],
                out_hbm.at[2 * h + j, pl.ds(s0 + c * CHUNK, CHUNK)],
                sem_w.at[slot]))
        return cps

    def start_pos(c, pslot):
        return pltpu.async_copy(
            pos_hbm.at[pl.ds(s0 + c * CHUNK, CHUNK)],
            pbuf.at[pslot], sem_p.at[pslot])

    pos_pending = start_pos(0, 0)
    gathers = [start_gather(0, 0), start_gather(1, 1)] + [None] * (NBUF - 2)
    writebacks = [None] * NBUF

    for p in range(NUM_PAIRS):
        m = msk_i[p, :]
        mmf = jnp.where(m != 0, 0.0, 1.0)
        for lane in range(PAIR_ROWS):
            mmul_x[p, lane, :] = jnp.broadcast_to(mmf[lane], (LANES,))

    for p in range(NUM_PAIRS):
        slot = p % NBUF
        if p % 2 == 0:
            c = p // 2
            pos_pending.wait()
            if c + 1 < NUM_CHUNKS:
                pos_pending = start_pos(c + 1, (c + 1) % 2)
        pslot = (p // 2) % 2
        if p + 2 < NUM_PAIRS:
            pf = (p + 2) % NBUF
            if writebacks[pf] is not None:
                for cp in writebacks[pf]:
                    cp.wait()
                writebacks[pf] = None
            gathers[pf] = start_gather(p + 2, pf)
        gathers[slot].wait()
        gathers[slot] = None

        def row_add(row, carry):
            mra = mmul_x[p, row, :]
            mrb = mmul_x[p, CHUNK + row, :]

            @plsc.parallel_loop(0, EMBED_DIM, step=LANES, unroll=4)
            def vec_add(v):
                vsl = pl.ds(v, LANES)
                pv = pbuf[pslot, row, vsl]
                ta = tbuf[slot, row, vsl]
                tb = tbuf[slot, CHUNK + row, vsl]
                tbuf[slot, row, vsl] = (ta + pv) * mra
                tbuf[slot, CHUNK + row, vsl] = (tb + pv) * mrb

            return carry

        lax.fori_loop(0, CHUNK, row_add, 0)
        writebacks[slot] = start_writebacks(p, slot)

    for wbs in writebacks:
        if wbs is not None:
            for cp in wbs:
                cp.wait()


@functools.partial(jax.jit, donate_argnums=())
def _embed(inputs, masks_i32, token_table, pos_flat):
    mesh = plsc.VectorSubcoreMesh(
        core_axis_name="c", subcore_axis_name="s",
        num_cores=NUM_CORES, num_subcores=NUM_SUBCORES)
    f = pl.kernel(
        _embed_body,
        out_type=jax.ShapeDtypeStruct(
            (BATCH, MAX_SEQ_LEN, EMBED_DIM), jnp.float32),
        mesh=mesh,
        scratch_types=[
            pltpu.VMEM((NUM_PAIRS, PAIR_ROWS), jnp.int32),
            pltpu.VMEM((NUM_PAIRS, PAIR_ROWS), jnp.int32),
            pltpu.VMEM((NUM_PAIRS, PAIR_ROWS, LANES), jnp.float32),
            pltpu.VMEM((2, CHUNK, EMBED_DIM), jnp.float32),
            pltpu.VMEM((NBUF, PAIR_ROWS, EMBED_DIM), jnp.float32),
            pltpu.SemaphoreType.DMA,
            pltpu.SemaphoreType.DMA((NBUF,)),
            pltpu.SemaphoreType.DMA((2,)),
            pltpu.SemaphoreType.DMA((NBUF,)),
        ],
    )
    return f(inputs, masks_i32, token_table, pos_flat)


def kernel(inputs, masks, token_table, pos_embedding):
    idx = inputs.astype(jnp.int32)
    msk = masks.astype(jnp.int32)
    pos_flat = pos_embedding.reshape(MAX_SEQ_LEN, EMBED_DIM)
    return _embed(idx, msk, token_table, pos_flat)
